# baseline (speedup 1.0000x reference)
import jax
import jax.numpy as jnp
import numpy as np
from jax.experimental import pallas as pl
from jax.experimental.pallas import tpu as pltpu

_TEMP = 0.1
_G = 8192
_S = 256
_T = 20
_C = _S * _T

_BG = 1024
_CH = 512
_NCH = _C // _CH
_NSTEPS = _G // _BG + 1

_KS0 = np.uint32(0)
_KS1 = np.uint32(42)
_KS2 = np.uint32(0 ^ 42 ^ 0x1BD11BDA)
_ROTS = ((13, 15, 26, 6), (17, 29, 16, 24))
_KSEQ = ((_KS1, _KS2, 1), (_KS2, _KS0, 2), (_KS0, _KS1, 3),
         (_KS1, _KS2, 4), (_KS2, _KS0, 5))


def _threefry_bits(x1):
    def rotl(x, d):
        return (x << np.uint32(d)) | (x >> np.uint32(32 - d))

    x0 = x1
    x1 = x0 ^ rotl(x1, _ROTS[0][0])
    for r in _ROTS[0][1:]:
        x0 = x0 + x1
        x1 = x0 ^ rotl(x1, r)
    a, b, c = _KSEQ[0]
    x0 = x0 + a
    x1 = x1 + np.uint32((int(b) + c) & 0xFFFFFFFF)
    for i in range(1, 5):
        for r in _ROTS[i % 2]:
            x0 = x0 + x1
            x1 = x0 ^ rotl(x1, r)
        a, b, c = _KSEQ[i]
        x0 = x0 + a
        x1 = x1 + np.uint32((int(b) + c) & 0xFFFFFFFF)
    return x0 ^ x1


def _sampler_kernel(delta_ref, oh_t_ref, out_ref, probs_ref, nw_ref, tok_ref,
                    macc_ref, jacc_ref):
    step = pl.program_id(0)

    @pl.when(step == 0)
    def _prologue():
        logits = delta_ref[...] * np.float32(1.0 / _TEMP)
        m = jnp.max(logits)
        e = jnp.exp(logits - m)
        probs_ref[...] = e / jnp.sum(e)
        nw_ref[...] = -jnp.exp(-logits)
        iota_t = jax.lax.broadcasted_iota(jnp.int32, (_T, _S), 0).astype(jnp.float32)
        tok_ref[...] = jnp.sum(oh_t_ref[...] * iota_t, axis=0,
                               keepdims=True).astype(jnp.int32)

    @pl.when(step > 0)
    def _extract_prev():
        best_m = macc_ref[...]
        best_j = jacc_ref[...]
        lane = jax.lax.broadcasted_iota(jnp.int32, (_BG, _CH), 1)
        colg = best_j * _CH + lane
        v = jnp.min(best_m, axis=1, keepdims=True)
        cand = jnp.where(best_m == v, colg, jnp.int32(2 ** 30))
        best_c = jnp.min(cand, axis=1, keepdims=True)

        res = best_c // _T
        aa = best_c - res * _T
        pos = jax.lax.broadcasted_iota(jnp.int32, (_BG, _S), 1)
        out_ref[...] = jnp.where(pos == res, aa, tok_ref[...])

    @pl.when(step < _NSTEPS - 1)
    def _sample():
        base = (step * (_BG * _C) + int(_KS1)).astype(jnp.uint32)
        pre = (base
               + jax.lax.broadcasted_iota(jnp.uint32, (_BG, _CH), 0) * np.uint32(_C)
               + jax.lax.broadcasted_iota(jnp.uint32, (_BG, _CH), 1))

        def chunk_val(j):
            off = j * _CH
            off = off.astype(jnp.uint32) if hasattr(off, "astype") else np.uint32(off)
            bits = _threefry_bits(pre + off)
            fb = (bits >> np.uint32(9)) | np.uint32(0x3F800000)
            u = jax.lax.bitcast_convert_type(fb, jnp.float32) - np.float32(1.0)
            nw = nw_ref[0:1, pl.ds(j * _CH, _CH)]
            return jnp.log(u) * nw

        macc_ref[...] = chunk_val(0)
        jacc_ref[...] = jnp.zeros((_BG, _CH), jnp.int32)

        def chunk_body(j, carry):
            val = chunk_val(j)
            best_m = macc_ref[...]
            upd = val < best_m
            macc_ref[...] = jnp.where(upd, val, best_m)
            jacc_ref[...] = jnp.where(upd, j, jacc_ref[...])
            return carry

        jax.lax.fori_loop(1, _NCH, chunk_body, 0)


@jax.jit
def kernel(seq_one_hot, delta_ij):
    delta2 = delta_ij.reshape(1, _C)
    oh_t = seq_one_hot.reshape(_S, _T).T

    mutated, probs = pl.pallas_call(
        _sampler_kernel,
        grid=(_NSTEPS,),
        in_specs=[
            pl.BlockSpec((1, _C), lambda i: (0, 0)),
            pl.BlockSpec((_T, _S), lambda i: (0, 0)),
        ],
        out_specs=[
            pl.BlockSpec((_BG, _S), lambda i: (jnp.maximum(i - 1, 0), 0)),
            pl.BlockSpec((1, _C), lambda i: (0, 0)),
        ],
        out_shape=[
            jax.ShapeDtypeStruct((_G, _S), jnp.int32),
            jax.ShapeDtypeStruct((1, _C), jnp.float32),
        ],
        scratch_shapes=[
            pltpu.VMEM((1, _C), jnp.float32),
            pltpu.VMEM((1, _S), jnp.int32),
            pltpu.VMEM((_BG, _CH), jnp.float32),
            pltpu.VMEM((_BG, _CH), jnp.int32),
        ],
        compiler_params=pltpu.CompilerParams(
            dimension_semantics=("arbitrary",),
        ),
    )(delta2, oh_t)
    return mutated, probs.reshape(_C)


# speedup vs baseline: 1.0076x; 1.0076x over previous
import jax
import jax.numpy as jnp
import numpy as np
from jax.experimental import pallas as pl
from jax.experimental.pallas import tpu as pltpu

_TEMP = 0.1
_G = 8192
_S = 256
_T = 20
_C = _S * _T

_BG = 512
_CH = 512
_NCH = _C // _CH
_NSTEPS = _G // _BG + 1

_KS0 = np.uint32(0)
_KS1 = np.uint32(42)
_KS2 = np.uint32(0 ^ 42 ^ 0x1BD11BDA)
_ROTS = ((13, 15, 26, 6), (17, 29, 16, 24))
_KSEQ = ((_KS1, _KS2, 1), (_KS2, _KS0, 2), (_KS0, _KS1, 3),
         (_KS1, _KS2, 4), (_KS2, _KS0, 5))


def _threefry_bits(x1):
    def rotl(x, d):
        return (x << np.uint32(d)) | (x >> np.uint32(32 - d))

    x0 = x1
    x1 = x0 ^ rotl(x1, _ROTS[0][0])
    for r in _ROTS[0][1:]:
        x0 = x0 + x1
        x1 = x0 ^ rotl(x1, r)
    a, b, c = _KSEQ[0]
    x0 = x0 + a
    x1 = x1 + np.uint32((int(b) + c) & 0xFFFFFFFF)
    for i in range(1, 5):
        for r in _ROTS[i % 2]:
            x0 = x0 + x1
            x1 = x0 ^ rotl(x1, r)
        a, b, c = _KSEQ[i]
        x0 = x0 + a
        x1 = x1 + np.uint32((int(b) + c) & 0xFFFFFFFF)
    return x0 ^ x1


def _sampler_kernel(delta_ref, oh_t_ref, out_ref, probs_ref, nw_ref, tok_ref,
                    macc_ref, jacc_ref):
    step = pl.program_id(0)

    @pl.when(step == 0)
    def _prologue():
        logits = delta_ref[...] * np.float32(1.0 / _TEMP)
        m = jnp.max(logits)
        e = jnp.exp(logits - m)
        probs_ref[...] = e / jnp.sum(e)
        nw_ref[...] = -jnp.exp(-logits)
        iota_t = jax.lax.broadcasted_iota(jnp.int32, (_T, _S), 0).astype(jnp.float32)
        tok_ref[...] = jnp.sum(oh_t_ref[...] * iota_t, axis=0,
                               keepdims=True).astype(jnp.int32)

    @pl.when(step > 0)
    def _extract_prev():
        best_m = macc_ref[...]
        best_j = jacc_ref[...]
        lane = jax.lax.broadcasted_iota(jnp.int32, (_BG, _CH), 1)
        colg = best_j * _CH + lane
        v = jnp.min(best_m, axis=1, keepdims=True)
        cand = jnp.where(best_m == v, colg, jnp.int32(2 ** 30))
        best_c = jnp.min(cand, axis=1, keepdims=True)

        res = best_c // _T
        aa = best_c - res * _T
        pos = jax.lax.broadcasted_iota(jnp.int32, (_BG, _S), 1)
        out_ref[...] = jnp.where(pos == res, aa, tok_ref[...])

    @pl.when(step < _NSTEPS - 1)
    def _sample():
        base = (step * (_BG * _C) + int(_KS1)).astype(jnp.uint32)
        pre = (base
               + jax.lax.broadcasted_iota(jnp.uint32, (_BG, _CH), 0) * np.uint32(_C)
               + jax.lax.broadcasted_iota(jnp.uint32, (_BG, _CH), 1))

        def chunk_val(j):
            off = j * _CH
            off = off.astype(jnp.uint32) if hasattr(off, "astype") else np.uint32(off)
            bits = _threefry_bits(pre + off)
            fb = (bits >> np.uint32(9)) | np.uint32(0x3F800000)
            u = jax.lax.bitcast_convert_type(fb, jnp.float32) - np.float32(1.0)
            nw = nw_ref[0:1, pl.ds(j * _CH, _CH)]
            return jnp.log(u) * nw

        macc_ref[...] = chunk_val(0)
        jacc_ref[...] = jnp.zeros((_BG, _CH), jnp.int32)

        def chunk_body(j, carry):
            val = chunk_val(j)
            best_m = macc_ref[...]
            upd = val < best_m
            macc_ref[...] = jnp.minimum(val, best_m)
            jacc_ref[...] = jnp.where(upd, j, jacc_ref[...])
            return carry

        jax.lax.fori_loop(1, _NCH, chunk_body, 0)


@jax.jit
def kernel(seq_one_hot, delta_ij):
    delta2 = delta_ij.reshape(1, _C)
    oh_t = seq_one_hot.reshape(_S, _T).T

    mutated, probs = pl.pallas_call(
        _sampler_kernel,
        grid=(_NSTEPS,),
        in_specs=[
            pl.BlockSpec((1, _C), lambda i: (0, 0)),
            pl.BlockSpec((_T, _S), lambda i: (0, 0)),
        ],
        out_specs=[
            pl.BlockSpec((_BG, _S), lambda i: (jnp.maximum(i - 1, 0), 0)),
            pl.BlockSpec((1, _C), lambda i: (0, 0)),
        ],
        out_shape=[
            jax.ShapeDtypeStruct((_G, _S), jnp.int32),
            jax.ShapeDtypeStruct((1, _C), jnp.float32),
        ],
        scratch_shapes=[
            pltpu.VMEM((1, _C), jnp.float32),
            pltpu.VMEM((1, _S), jnp.int32),
            pltpu.VMEM((_BG, _CH), jnp.float32),
            pltpu.VMEM((_BG, _CH), jnp.int32),
        ],
        compiler_params=pltpu.CompilerParams(
            dimension_semantics=("arbitrary",),
        ),
    )(delta2, oh_t)
    return mutated, probs.reshape(_C)


# speedup vs baseline: 1.0154x; 1.0078x over previous
import jax
import jax.numpy as jnp
import numpy as np
from jax.experimental import pallas as pl
from jax.experimental.pallas import tpu as pltpu

_TEMP = 0.1
_G = 8192
_S = 256
_T = 20
_C = _S * _T

_BG = 512
_CH = 512
_NCH = _C // _CH
_NSTEPS = _G // _BG + 1

_KS0 = np.uint32(0)
_KS1 = np.uint32(42)
_KS2 = np.uint32(0 ^ 42 ^ 0x1BD11BDA)
_ROTS = ((13, 15, 26, 6), (17, 29, 16, 24))
_KSEQ = ((_KS1, _KS2, 1), (_KS2, _KS0, 2), (_KS0, _KS1, 3),
         (_KS1, _KS2, 4), (_KS2, _KS0, 5))


def _threefry_bits(x1):
    def rotl(x, d):
        return (x << np.uint32(d)) | (x >> np.uint32(32 - d))

    x0 = x1
    x1 = x0 ^ rotl(x1, _ROTS[0][0])
    for r in _ROTS[0][1:]:
        x0 = x0 + x1
        x1 = x0 ^ rotl(x1, r)
    a, b, c = _KSEQ[0]
    x0 = x0 + a
    x1 = x1 + np.uint32((int(b) + c) & 0xFFFFFFFF)
    for i in range(1, 5):
        for r in _ROTS[i % 2]:
            x0 = x0 + x1
            x1 = x0 ^ rotl(x1, r)
        a, b, c = _KSEQ[i]
        x0 = x0 + a
        x1 = x1 + np.uint32((int(b) + c) & 0xFFFFFFFF)
    return x0 ^ x1


def _sampler_kernel(delta_ref, oh_t_ref, out_ref, probs_ref, nw_ref, tok_ref,
                    macc_ref, jacc_ref):
    step = pl.program_id(0)

    @pl.when(step == 0)
    def _prologue():
        logits = delta_ref[...] * np.float32(1.0 / _TEMP)
        m = jnp.max(logits)
        e = jnp.exp(logits - m)
        probs_ref[...] = e / jnp.sum(e)
        nw_ref[...] = -jnp.exp(-logits)
        iota_t = jax.lax.broadcasted_iota(jnp.int32, (_T, _S), 0).astype(jnp.float32)
        tok_ref[...] = jnp.sum(oh_t_ref[...] * iota_t, axis=0,
                               keepdims=True).astype(jnp.int32)

    @pl.when(step > 0)
    def _extract_prev():
        best_m = macc_ref[...]
        best_j = jacc_ref[...]
        lane = jax.lax.broadcasted_iota(jnp.int32, (_BG, _CH), 1)
        colg = best_j * _CH + lane
        v = jnp.min(best_m, axis=1, keepdims=True)
        cand = jnp.where(best_m == v, colg, jnp.int32(2 ** 30))
        best_c = jnp.min(cand, axis=1, keepdims=True)

        res = best_c // _T
        aa = best_c - res * _T
        pos = jax.lax.broadcasted_iota(jnp.int32, (_BG, _S), 1)
        out_ref[...] = jnp.where(pos == res, aa, tok_ref[...])

    @pl.when(step < _NSTEPS - 1)
    def _sample():
        base = (step * (_BG * _C) + int(_KS1)).astype(jnp.uint32)
        pre = (base
               + jax.lax.broadcasted_iota(jnp.uint32, (_BG, _CH), 0) * np.uint32(_C)
               + jax.lax.broadcasted_iota(jnp.uint32, (_BG, _CH), 1))

        def chunk_val(j):
            off = j * _CH
            off = off.astype(jnp.uint32) if hasattr(off, "astype") else np.uint32(off)
            bits = _threefry_bits(pre + off)
            fb = (bits >> np.uint32(9)) | np.uint32(0x3F800000)
            u = jax.lax.bitcast_convert_type(fb, jnp.float32) - np.float32(1.0)
            nw = nw_ref[0:1, pl.ds(j * _CH, _CH)]
            return jnp.log(u) * nw

        macc_ref[...] = chunk_val(0)
        jacc_ref[...] = jnp.zeros((_BG, _CH), jnp.int32)

        def chunk_body(i, carry):
            j0 = 1 + i * 3
            v0 = chunk_val(j0)
            v1 = chunk_val(j0 + 1)
            v2 = chunk_val(j0 + 2)
            v01 = jnp.minimum(v0, v1)
            j01 = jnp.where(v1 < v0, j0 + 1, j0)
            v012 = jnp.minimum(v01, v2)
            j012 = jnp.where(v2 < v01, j0 + 2, j01)
            best_m = macc_ref[...]
            upd = v012 < best_m
            macc_ref[...] = jnp.minimum(v012, best_m)
            jacc_ref[...] = jnp.where(upd, j012, jacc_ref[...])
            return carry

        jax.lax.fori_loop(0, (_NCH - 1) // 3, chunk_body, 0)


@jax.jit
def kernel(seq_one_hot, delta_ij):
    delta2 = delta_ij.reshape(1, _C)
    oh_t = seq_one_hot.reshape(_S, _T).T

    mutated, probs = pl.pallas_call(
        _sampler_kernel,
        grid=(_NSTEPS,),
        in_specs=[
            pl.BlockSpec((1, _C), lambda i: (0, 0)),
            pl.BlockSpec((_T, _S), lambda i: (0, 0)),
        ],
        out_specs=[
            pl.BlockSpec((_BG, _S), lambda i: (jnp.maximum(i - 1, 0), 0)),
            pl.BlockSpec((1, _C), lambda i: (0, 0)),
        ],
        out_shape=[
            jax.ShapeDtypeStruct((_G, _S), jnp.int32),
            jax.ShapeDtypeStruct((1, _C), jnp.float32),
        ],
        scratch_shapes=[
            pltpu.VMEM((1, _C), jnp.float32),
            pltpu.VMEM((1, _S), jnp.int32),
            pltpu.VMEM((_BG, _CH), jnp.float32),
            pltpu.VMEM((_BG, _CH), jnp.int32),
        ],
        compiler_params=pltpu.CompilerParams(
            dimension_semantics=("arbitrary",),
        ),
    )(delta2, oh_t)
    return mutated, probs.reshape(_C)
